# Initial kernel scaffold; baseline (speedup 1.0000x reference)
import jax
import jax.numpy as jnp
from jax.experimental import pallas as pl


def kernel(x, edge_index, edge_attr, W_0, Wf_0, W_1, Wf_1, W_2, Wf_2, W_3, Wf_3, fc_w, fc_b):
    raise NotImplementedError("write your pallas kernel here")


# speedup vs baseline: 3.2309x; 3.2309x over previous
import functools

import jax
import jax.numpy as jnp
from jax import lax
from jax.experimental import pallas as pl
from jax.experimental.pallas import tpu as pltpu
from jax.experimental.pallas import tpu_sc as plsc

_F32 = jnp.float32


def _mm_body(h_ref, w_ref, o_ref):
    o_ref[...] = jnp.dot(h_ref[...], w_ref[...], preferred_element_type=_F32)


def _mm(h, w):
    n, _ = h.shape
    _, hdim = w.shape
    return pl.pallas_call(
        _mm_body,
        out_shape=jax.ShapeDtypeStruct((n, hdim), _F32),
    )(h, w)


def _relu_mm_body(parts_ref, w_ref, o_ref):
    n = parts_ref.shape[0] // 2
    h = jnp.maximum(parts_ref[:n, :] + parts_ref[n:, :], 0.0)
    o_ref[...] = jnp.dot(h, w_ref[...], preferred_element_type=_F32)


def _relu_mm(parts, w):
    n = parts.shape[0] // 2
    hdim = w.shape[1]
    return pl.pallas_call(
        _relu_mm_body,
        out_shape=jax.ShapeDtypeStruct((n, hdim), _F32),
    )(parts, w)


def _filt_body(ea_ref, wf_ref, o_ref):
    o_ref[...] = jnp.dot(ea_ref[...], wf_ref[...], preferred_element_type=_F32)


def _filt(edge_attr, wf):
    e, de = edge_attr.shape
    hdim = wf.shape[1]
    blk = 4000
    grid = e // blk
    return pl.pallas_call(
        _filt_body,
        grid=(grid,),
        in_specs=[
            pl.BlockSpec((blk, de), lambda i: (i, 0)),
            pl.BlockSpec((de, hdim), lambda i: (0, 0)),
        ],
        out_specs=pl.BlockSpec((blk, hdim), lambda i: (i, 0)),
        out_shape=jax.ShapeDtypeStruct((e, hdim), _F32),
    )(edge_attr, wf)


def _final_body(parts_ref, fcw_ref, fcb_ref, o_ref):
    n = parts_ref.shape[0] // 2
    h = jnp.maximum(parts_ref[:n, :] + parts_ref[n:, :], 0.0)
    pooled = jnp.mean(h, axis=0, keepdims=True)
    logits = jnp.dot(pooled, fcw_ref[...], preferred_element_type=_F32)
    logits = logits + fcb_ref[...]
    m = jnp.max(logits, axis=1, keepdims=True)
    s = logits - m
    lse = jnp.log(jnp.sum(jnp.exp(s), axis=1, keepdims=True))
    o_ref[...] = s - lse


def _final(parts, fc_w, fc_b):
    c = fc_w.shape[1]
    return pl.pallas_call(
        _final_body,
        out_shape=jax.ShapeDtypeStruct((1, c), _F32),
    )(parts, fc_w, fc_b.reshape(1, c))


@functools.lru_cache(maxsize=None)
def _make_sc_scatter(n, e, hdim):
    info = plsc.get_sparse_core_info()
    nc, ns = info.num_cores, info.num_subcores
    nw = nc * ns
    ch = 128
    n_chunks = e // ch
    assert n_chunks * ch == e
    base_trips = n_chunks // nw
    extra = n_chunks - base_trips * nw
    cr = 80
    n_rchunks = n // cr
    assert n_rchunks * cr == n
    base_r = n_rchunks // ns
    extra_r = n_rchunks - base_r * ns
    lanes = 16
    mesh = plsc.VectorSubcoreMesh(core_axis_name="c", subcore_axis_name="s")

    @functools.partial(
        pl.kernel,
        out_type=jax.ShapeDtypeStruct((nc * n, hdim), _F32),
        mesh=mesh,
        scratch_types=[
            pltpu.VMEM((ch,), jnp.int32),
            pltpu.VMEM((ch,), jnp.int32),
            pltpu.VMEM((ch, hdim), _F32),
            pltpu.VMEM((ch, hdim), _F32),
            pltpu.VMEM((cr, hdim), _F32),
            pltpu.VMEM_SHARED((n, hdim), _F32),
            pltpu.SemaphoreType.DMA,
        ],
    )
    def sc_scatter(hw_hbm, filt_hbm, src_hbm, dst_hbm, out_hbm,
                   src_v, dst_v, rows_v, filt_v, zero_v, acc_sp, sem):
        c = lax.axis_index("c")
        s = lax.axis_index("s")
        wid = s * nc + c

        def zfill_row(i, _):
            def zfill_col(j, _):
                zero_v[i, pl.ds(j * lanes, lanes)] = jnp.zeros((lanes,), _F32)
                return 0
            return lax.fori_loop(0, hdim // lanes, zfill_col, 0)
        lax.fori_loop(0, cr, zfill_row, 0)
        rtrips = base_r + jnp.where(s < extra_r, 1, 0)

        def zero_body(k, _):
            roff = (s + k * ns) * cr
            pltpu.sync_copy(zero_v, acc_sp.at[pl.ds(roff, cr)])
            return 0
        lax.fori_loop(0, rtrips, zero_body, 0)
        plsc.subcore_barrier()

        trips = base_trips + jnp.where(wid < extra, 1, 0)

        def body(t, _):
            off = (wid + t * nw) * ch
            pltpu.sync_copy(src_hbm.at[pl.ds(off, ch)], src_v)
            pltpu.sync_copy(dst_hbm.at[pl.ds(off, ch)], dst_v)
            pltpu.async_copy(hw_hbm.at[src_v], rows_v, sem).wait()
            pltpu.sync_copy(filt_hbm.at[pl.ds(off, ch)], filt_v)

            def mul_row(i, _):
                def mul_col(j, _):
                    sl = pl.ds(j * lanes, lanes)
                    rows_v[i, sl] = rows_v[i, sl] * filt_v[i, sl]
                    return 0
                return lax.fori_loop(0, hdim // lanes, mul_col, 0)
            lax.fori_loop(0, ch, mul_row, 0)

            pltpu.sync_copy(rows_v, acc_sp.at[dst_v], add=True)
            return 0
        lax.fori_loop(0, trips, body, 0)

        plsc.subcore_barrier()

        def out_body(k, _):
            roff = (s + k * ns) * cr
            pltpu.sync_copy(acc_sp.at[pl.ds(roff, cr)],
                            out_hbm.at[pl.ds(c * n + roff, cr)])
            return 0
        lax.fori_loop(0, rtrips, out_body, 0)

    return sc_scatter


def kernel(x, edge_index, edge_attr, W_0, Wf_0, W_1, Wf_1, W_2, Wf_2,
           W_3, Wf_3, fc_w, fc_b):
    n, _ = x.shape
    e = edge_attr.shape[0]
    hdim = W_0.shape[1]
    src = edge_index[0]
    dst = edge_index[1]
    sc_scatter = _make_sc_scatter(n, e, hdim)

    ws = [W_0, W_1, W_2, W_3]
    wfs = [Wf_0, Wf_1, Wf_2, Wf_3]
    parts = None
    for i in range(4):
        hw = _mm(x, ws[i]) if i == 0 else _relu_mm(parts, ws[i])
        filt = _filt(edge_attr, wfs[i])
        parts = sc_scatter(hw, filt, src, dst)
    return _final(parts, fc_w, fc_b)
